# Initial kernel scaffold; baseline (speedup 1.0000x reference)
import jax
import jax.numpy as jnp
from jax.experimental import pallas as pl


def kernel(h, edge_index, W1, attn_l1, attn_r1, b1, W2, attn_l2, attn_r2, b2):
    raise NotImplementedError("write your pallas kernel here")


# speedup vs baseline: 4.9844x; 4.9844x over previous
import functools

import jax
import jax.numpy as jnp
from jax import lax
from jax.experimental import pallas as pl
from jax.experimental.pallas import tpu as pltpu
from jax.experimental.pallas import tpu_sc as plsc

_N = 10000
_NP = 10240
_E = 320000
_IN = 128
_HID = 64
_OUT = 128
_HEADS = 8

_NC = 2
_NS = 16
_G = 80


def _tc_feat(x, W, AL, AR, elu_bias=None):
    NPAD, K = x.shape
    M = W.shape[1]
    BN = 1280
    grid = (NPAD // BN,)
    with_elu = elu_bias is not None

    def body(*refs):
        if with_elu:
            x_ref, b_ref, w_ref, al_ref, ar_ref, f_ref, l_ref, r_ref = refs
            xv = x_ref[...] + b_ref[...]
            xv = jnp.where(xv > 0, xv, jnp.exp(xv) - 1.0)
        else:
            x_ref, w_ref, al_ref, ar_ref, f_ref, l_ref, r_ref = refs
            xv = x_ref[...]
        f = jnp.dot(xv, w_ref[...], preferred_element_type=jnp.float32)
        f_ref[...] = f
        l_ref[...] = jnp.dot(f, al_ref[...], preferred_element_type=jnp.float32)
        r_ref[...] = jnp.dot(f, ar_ref[...], preferred_element_type=jnp.float32)

    in_specs = [pl.BlockSpec((BN, K), lambda i: (i, 0))]
    args = [x]
    if with_elu:
        in_specs.append(pl.BlockSpec((1, K), lambda i: (0, 0)))
        args.append(elu_bias.reshape(1, K))
    in_specs += [
        pl.BlockSpec((K, M), lambda i: (0, 0)),
        pl.BlockSpec((M, 16), lambda i: (0, 0)),
        pl.BlockSpec((M, 16), lambda i: (0, 0)),
    ]
    args += [W, AL, AR]
    return pl.pallas_call(
        body,
        grid=grid,
        in_specs=in_specs,
        out_specs=[
            pl.BlockSpec((BN, M), lambda i: (i, 0)),
            pl.BlockSpec((BN, 16), lambda i: (i, 0)),
            pl.BlockSpec((BN, 16), lambda i: (i, 0)),
        ],
        out_shape=[
            jax.ShapeDtypeStruct((NPAD, M), jnp.float32),
            jax.ShapeDtypeStruct((NPAD, 16), jnp.float32),
            jax.ShapeDtypeStruct((NPAD, 16), jnp.float32),
        ],
    )(*args)


def _tc_bias(x, b):
    NPAD, M = x.shape
    BN = 1280

    def body(x_ref, b_ref, o_ref):
        o_ref[...] = x_ref[...] + b_ref[...]

    return pl.pallas_call(
        body,
        grid=(NPAD // BN,),
        in_specs=[pl.BlockSpec((BN, M), lambda i: (i, 0)),
                  pl.BlockSpec((1, M), lambda i: (0, 0))],
        out_specs=pl.BlockSpec((BN, M), lambda i: (i, 0)),
        out_shape=jax.ShapeDtypeStruct((NPAD, M), jnp.float32),
    )(x, b.reshape(1, M))


def _sc_stats(eL, eR, src, dst):
    EPT = _E // (_NC * _NS)
    NG = EPT // _G
    RT = _NP // _NS
    mesh = plsc.VectorSubcoreMesh(core_axis_name="c", subcore_axis_name="s")

    def body(eL_hbm, eR_hbm, src_hbm, dst_hbm, ex_hbm, sp_hbm,
             s_sh, src_i, dst_i, lrows, rrows, exs, zbuf):
        cid = lax.axis_index("c")
        sid = lax.axis_index("s")
        wid = cid * _NS + sid
        z16 = jnp.zeros((16,), jnp.float32)

        def zrow(i, _):
            zbuf[i, :] = z16
            return 0
        lax.fori_loop(0, RT, zrow, 0)
        pltpu.sync_copy(zbuf, s_sh.at[pl.ds(sid * RT, RT)])
        plsc.subcore_barrier()

        def group(g, _):
            e0 = wid * EPT + g * _G
            pltpu.sync_copy(src_hbm.at[pl.ds(e0, _G)], src_i)
            pltpu.sync_copy(dst_hbm.at[pl.ds(e0, _G)], dst_i)
            pltpu.sync_copy(eL_hbm.at[src_i], lrows)
            pltpu.sync_copy(eR_hbm.at[dst_i], rrows)

            def erow(e, _):
                v = lrows[e, :] + rrows[e, :]
                v = jnp.where(v >= 0, v, 0.2 * v)
                exs[e, :] = jnp.exp(v)
                return 0
            lax.fori_loop(0, _G, erow, 0)
            pltpu.sync_copy(exs, ex_hbm.at[pl.ds(e0, _G)])
            pltpu.sync_copy(exs, s_sh.at[dst_i], add=True)
            return 0
        lax.fori_loop(0, NG, group, 0)
        plsc.subcore_barrier()

        pltpu.sync_copy(s_sh.at[pl.ds(sid * RT, RT)], zbuf)
        pltpu.sync_copy(zbuf, sp_hbm.at[pl.ds(cid * _NP + sid * RT, RT)])

    return pl.kernel(
        body,
        out_type=(jax.ShapeDtypeStruct((_E, 16), jnp.float32),
                  jax.ShapeDtypeStruct((2 * _NP, 16), jnp.float32)),
        mesh=mesh,
        compiler_params=pltpu.CompilerParams(use_tc_tiling_on_sc=False),
        scratch_types=(
            pltpu.VMEM_SHARED((_NP, 16), jnp.float32),
            pltpu.VMEM((_G,), jnp.int32),
            pltpu.VMEM((_G,), jnp.int32),
            pltpu.VMEM((_G, 16), jnp.float32),
            pltpu.VMEM((_G, 16), jnp.float32),
            pltpu.VMEM((_G, 16), jnp.float32),
            pltpu.VMEM((RT, 16), jnp.float32),
        ),
    )(eL, eR, src, dst)


def _sc_agg(W, M, R, NH, feat, src, dst, ex, s0, s1):
    HIDW = W // NH
    NV = HIDW // 16
    GA = 32
    EPT = _E // _NS
    NG = EPT // GA
    RT = M // _NS
    NZ = RT // 16
    mesh = plsc.VectorSubcoreMesh(core_axis_name="c", subcore_axis_name="s")

    def body(feat_hbm, src_hbm, dst_hbm, ex_hbm, s0_hbm, s1_hbm, out_hbm,
             acc_sh, src_i, dst_i, featb, exb, s0b, s1b, alphab,
             msgb, zrb):
        cid = lax.axis_index("c")
        sid = lax.axis_index("s")
        z16 = jnp.zeros((16,), jnp.float32)

        for r in range(R):
            base = (cid * R + r) * M

            def zrow(i, _):
                for k in range(W // 16):
                    zrb[i, pl.ds(k * 16, 16)] = z16
                return 0
            lax.fori_loop(0, 16, zrow, 0)

            def initk(k, _):
                pltpu.sync_copy(zrb, acc_sh.at[pl.ds(sid * RT + k * 16, 16)])
                return 0
            lax.fori_loop(0, NZ, initk, 0)
            plsc.subcore_barrier()

            def group(g, _):
                e0 = sid * EPT + g * GA
                pltpu.sync_copy(src_hbm.at[pl.ds(e0, GA)], src_i)
                pltpu.sync_copy(dst_hbm.at[pl.ds(e0, GA)], dst_i)
                pltpu.sync_copy(feat_hbm.at[src_i], featb)
                pltpu.sync_copy(ex_hbm.at[pl.ds(e0, GA)], exb)
                pltpu.sync_copy(s0_hbm.at[dst_i], s0b)
                pltpu.sync_copy(s1_hbm.at[dst_i], s1b)

                def arow(e, _):
                    alphab[e, :] = exb[e, :] / (s0b[e, :] + s1b[e, :])
                    return 0
                lax.fori_loop(0, GA, arow, 0)

                for c in range(GA // 16):
                    dv = dst_i[pl.ds(c * 16, 16)]
                    inr = (dv >= base) & (dv < base + M)
                    doff = jnp.where(inr, dv - base, M)

                    def ebody(t, _):
                        eidx = c * 16 + t
                        ar = alphab[eidx, :]
                        for h in range(NH):
                            a = ar[h]
                            for k in range(NV):
                                off = h * HIDW + k * 16
                                msgb[t, pl.ds(off, 16)] = (
                                    a * featb[eidx, pl.ds(off, 16)])
                        return 0
                    lax.fori_loop(0, 16, ebody, 0)
                    pltpu.sync_copy(msgb, acc_sh.at[doff], add=True)
                return 0
            lax.fori_loop(0, NG, group, 0)
            plsc.subcore_barrier()

            def rok(k, _):
                row0 = sid * RT + k * 16
                pltpu.sync_copy(acc_sh.at[pl.ds(row0, 16)], zrb)
                pltpu.sync_copy(zrb, out_hbm.at[pl.ds(base + row0, 16)])
                return 0
            lax.fori_loop(0, NZ, rok, 0)
            plsc.subcore_barrier()

    return pl.kernel(
        body,
        out_type=jax.ShapeDtypeStruct((_NP, W), jnp.float32),
        mesh=mesh,
        compiler_params=pltpu.CompilerParams(use_tc_tiling_on_sc=False),
        scratch_types=(
            pltpu.VMEM_SHARED((M + 16, W), jnp.float32),
            pltpu.VMEM((GA,), jnp.int32),
            pltpu.VMEM((GA,), jnp.int32),
            pltpu.VMEM((GA, W), jnp.float32),
            pltpu.VMEM((GA, 16), jnp.float32),
            pltpu.VMEM((GA, 16), jnp.float32),
            pltpu.VMEM((GA, 16), jnp.float32),
            pltpu.VMEM((GA, 16), jnp.float32),
            pltpu.VMEM((16, W), jnp.float32),
            pltpu.VMEM((16, W), jnp.float32),
        ),
    )(feat, src, dst, ex, s0, s1)


def kernel(h, edge_index, W1, attn_l1, attn_r1, b1, W2, attn_l2, attn_r2, b2):
    src = edge_index[0]
    dst = edge_index[1]
    h_pad = jnp.zeros((_NP, _IN), jnp.float32).at[:_N].set(h)

    eye = jnp.eye(_HEADS, dtype=jnp.float32)
    Al = jnp.einsum("hd,hk->hdk", attn_l1[0], eye).reshape(_HEADS * _HID, _HEADS)
    Ar = jnp.einsum("hd,hk->hdk", attn_r1[0], eye).reshape(_HEADS * _HID, _HEADS)
    AL1 = jnp.concatenate([Al, Al], axis=1)
    AR1 = jnp.concatenate([Ar, Ar], axis=1)
    AL2 = jnp.broadcast_to(attn_l2[0, 0][:, None], (_OUT, 16))
    AR2 = jnp.broadcast_to(attn_r2[0, 0][:, None], (_OUT, 16))

    feat1, eL1, eR1 = _tc_feat(h_pad, W1, AL1, AR1)
    ex1, sp1 = _sc_stats(eL1, eR1, src, dst)
    out1 = _sc_agg(_HEADS * _HID, 2560, 2, _HEADS,
                   feat1, src, dst, ex1, sp1[:_NP], sp1[_NP:])

    feat2, eL2, eR2 = _tc_feat(out1, W2, AL2, AR2, elu_bias=b1)
    ex2, sp2 = _sc_stats(eL2, eR2, src, dst)
    out2 = _sc_agg(_OUT, 5120, 1, 1,
                   feat2, src, dst, ex2, sp2[:_NP], sp2[_NP:])

    final = _tc_bias(out2, b2)
    return final[:_N]


# speedup vs baseline: 6.0262x; 1.2090x over previous
import functools

import jax
import jax.numpy as jnp
from jax import lax
from jax.experimental import pallas as pl
from jax.experimental.pallas import tpu as pltpu
from jax.experimental.pallas import tpu_sc as plsc

_N = 10000
_NP = 10240
_E = 320000
_IN = 128
_HID = 64
_OUT = 128
_HEADS = 8

_NC = 2
_NS = 16
_G = 80


def _tc_feat(x, W, AL, AR, elu_bias=None):
    NPAD, K = x.shape
    M = W.shape[1]
    BN = 1280
    grid = (NPAD // BN,)
    with_elu = elu_bias is not None

    def body(*refs):
        if with_elu:
            x_ref, b_ref, w_ref, al_ref, ar_ref, f_ref, l_ref, r_ref = refs
            xv = x_ref[...] + b_ref[...]
            xv = jnp.where(xv > 0, xv, jnp.exp(xv) - 1.0)
        else:
            x_ref, w_ref, al_ref, ar_ref, f_ref, l_ref, r_ref = refs
            xv = x_ref[...]
        f = jnp.dot(xv, w_ref[...], preferred_element_type=jnp.float32)
        f_ref[...] = f
        l_ref[...] = jnp.dot(f, al_ref[...], preferred_element_type=jnp.float32)
        r_ref[...] = jnp.dot(f, ar_ref[...], preferred_element_type=jnp.float32)

    in_specs = [pl.BlockSpec((BN, K), lambda i: (i, 0))]
    args = [x]
    if with_elu:
        in_specs.append(pl.BlockSpec((1, K), lambda i: (0, 0)))
        args.append(elu_bias.reshape(1, K))
    in_specs += [
        pl.BlockSpec((K, M), lambda i: (0, 0)),
        pl.BlockSpec((M, 16), lambda i: (0, 0)),
        pl.BlockSpec((M, 16), lambda i: (0, 0)),
    ]
    args += [W, AL, AR]
    return pl.pallas_call(
        body,
        grid=grid,
        in_specs=in_specs,
        out_specs=[
            pl.BlockSpec((BN, M), lambda i: (i, 0)),
            pl.BlockSpec((BN, 16), lambda i: (i, 0)),
            pl.BlockSpec((BN, 16), lambda i: (i, 0)),
        ],
        out_shape=[
            jax.ShapeDtypeStruct((NPAD, M), jnp.float32),
            jax.ShapeDtypeStruct((NPAD, 16), jnp.float32),
            jax.ShapeDtypeStruct((NPAD, 16), jnp.float32),
        ],
    )(*args)


def _tc_bias(x, b):
    NPAD, M = x.shape
    BN = 1280

    def body(x_ref, b_ref, o_ref):
        o_ref[...] = x_ref[...] + b_ref[...]

    return pl.pallas_call(
        body,
        grid=(NPAD // BN,),
        in_specs=[pl.BlockSpec((BN, M), lambda i: (i, 0)),
                  pl.BlockSpec((1, M), lambda i: (0, 0))],
        out_specs=pl.BlockSpec((BN, M), lambda i: (i, 0)),
        out_shape=jax.ShapeDtypeStruct((NPAD, M), jnp.float32),
    )(x, b.reshape(1, M))


def _sc_stats(eL, eR, src, dst):
    EPT = _E // (_NC * _NS)
    NG = EPT // _G
    RT = _NP // _NS
    mesh = plsc.VectorSubcoreMesh(core_axis_name="c", subcore_axis_name="s")

    def body(eL_hbm, eR_hbm, src_hbm, dst_hbm, ex_hbm, sp_hbm,
             s_sh, src_i, dst_i, lrows, rrows, exs, zbuf):
        cid = lax.axis_index("c")
        sid = lax.axis_index("s")
        wid = cid * _NS + sid
        z16 = jnp.zeros((16,), jnp.float32)

        def zrow(i, _):
            zbuf[i, :] = z16
            return 0
        lax.fori_loop(0, RT, zrow, 0)
        pltpu.sync_copy(zbuf, s_sh.at[pl.ds(sid * RT, RT)])
        plsc.subcore_barrier()

        def group(g, _):
            e0 = wid * EPT + g * _G
            pltpu.sync_copy(src_hbm.at[pl.ds(e0, _G)], src_i)
            pltpu.sync_copy(dst_hbm.at[pl.ds(e0, _G)], dst_i)
            pltpu.sync_copy(eL_hbm.at[src_i], lrows)
            pltpu.sync_copy(eR_hbm.at[dst_i], rrows)

            def erow(e, _):
                v = lrows[e, :] + rrows[e, :]
                v = jnp.where(v >= 0, v, 0.2 * v)
                exs[e, :] = jnp.exp(v)
                return 0
            lax.fori_loop(0, _G, erow, 0)
            pltpu.sync_copy(exs, ex_hbm.at[pl.ds(e0, _G)])
            pltpu.sync_copy(exs, s_sh.at[dst_i], add=True)
            return 0
        lax.fori_loop(0, NG, group, 0)
        plsc.subcore_barrier()

        pltpu.sync_copy(s_sh.at[pl.ds(sid * RT, RT)], zbuf)
        pltpu.sync_copy(zbuf, sp_hbm.at[pl.ds(cid * _NP + sid * RT, RT)])

    return pl.kernel(
        body,
        out_type=(jax.ShapeDtypeStruct((_E, 16), jnp.float32),
                  jax.ShapeDtypeStruct((2 * _NP, 16), jnp.float32)),
        mesh=mesh,
        compiler_params=pltpu.CompilerParams(use_tc_tiling_on_sc=False),
        scratch_types=(
            pltpu.VMEM_SHARED((_NP, 16), jnp.float32),
            pltpu.VMEM((_G,), jnp.int32),
            pltpu.VMEM((_G,), jnp.int32),
            pltpu.VMEM((_G, 16), jnp.float32),
            pltpu.VMEM((_G, 16), jnp.float32),
            pltpu.VMEM((_G, 16), jnp.float32),
            pltpu.VMEM((RT, 16), jnp.float32),
        ),
    )(eL, eR, src, dst)


def _sc_agg(W, M, R, NH, feat, src, dst, ex, s0, s1):
    HIDW = W // NH
    NV = HIDW // 16
    EPT = _E // _NS
    CH = 4000
    NCH = EPT // CH
    NGC = CH // 16
    RT = M // _NS
    NZ = RT // 16
    mesh = plsc.VectorSubcoreMesh(core_axis_name="c", subcore_axis_name="s")

    def body(feat_hbm, src_hbm, dst_hbm, ex_hbm, s0_hbm, s1_hbm, out_hbm,
             acc_sh, src_big, dst_big, featA, featB, exA, exB, s0A, s0B,
             s1A, s1B, msgA, msgB, semGA, semGB, semSA, semSB):
        cid = lax.axis_index("c")
        sid = lax.axis_index("s")
        z16 = jnp.zeros((16,), jnp.float32)
        gM = jnp.full((16,), M, jnp.int32)

        def fire(gloc, e0k, featX, exX, s0X, s1X, semX):
            e0 = e0k + gloc * 16
            svec = src_big[pl.ds(gloc * 16, 16)]
            dvec = dst_big[pl.ds(gloc * 16, 16)]
            pltpu.async_copy(feat_hbm.at[svec], featX, semX)
            pltpu.async_copy(ex_hbm.at[pl.ds(e0, 16)], exX, semX)
            pltpu.async_copy(s0_hbm.at[dvec], s0X, semX)
            pltpu.async_copy(s1_hbm.at[dvec], s1X, semX)

        def waitg(featX, exX, s0X, s1X, semX):
            pltpu.make_async_copy(feat_hbm.at[gM], featX, semX).wait()
            pltpu.make_async_copy(ex_hbm.at[pl.ds(0, 16)], exX, semX).wait()
            pltpu.make_async_copy(s0_hbm.at[gM], s0X, semX).wait()
            pltpu.make_async_copy(s1_hbm.at[gM], s1X, semX).wait()

        def compute(gloc, base, featX, exX, s0X, s1X, msgX, semSX):
            dvec = dst_big[pl.ds(gloc * 16, 16)]
            inr = (dvec >= base) & (dvec < base + M)
            doff = jnp.where(inr, dvec - base, M)
            pltpu.make_async_copy(msgX, acc_sh.at[gM], semSX).wait()

            def ebody(t, _):
                ar = exX[t, :] / (s0X[t, :] + s1X[t, :])
                for h in range(NH):
                    a = ar[h]
                    for k in range(NV):
                        off = h * HIDW + k * 16
                        msgX[t, pl.ds(off, 16)] = (
                            a * featX[t, pl.ds(off, 16)])
                return 0
            lax.fori_loop(0, 16, ebody, 0)
            pltpu.async_copy(msgX, acc_sh.at[doff], semSX, add=True)

        bufA = (featA, exA, s0A, s1A)
        bufB = (featB, exB, s0B, s1B)

        for r in range(R):
            base = (cid * R + r) * M

            def zrow(i, _):
                for k in range(W // 16):
                    msgA[i, pl.ds(k * 16, 16)] = z16
                    msgB[i, pl.ds(k * 16, 16)] = z16
                return 0
            lax.fori_loop(0, 16, zrow, 0)

            def initk(k, _):
                pltpu.sync_copy(msgA, acc_sh.at[pl.ds(sid * RT + k * 16, 16)])
                return 0
            lax.fori_loop(0, NZ, initk, 0)
            plsc.subcore_barrier()

            pltpu.async_copy(msgA, acc_sh.at[gM], semSA, add=True)
            pltpu.async_copy(msgB, acc_sh.at[gM], semSB, add=True)

            def chunk(k, _):
                e0k = sid * EPT + k * CH
                pltpu.sync_copy(src_hbm.at[pl.ds(e0k, CH)], src_big)
                pltpu.sync_copy(dst_hbm.at[pl.ds(e0k, CH)], dst_big)
                fire(0, e0k, *bufA, semGA)

                def pair(p, _):
                    g = 2 * p
                    fire(g + 1, e0k, *bufB, semGB)
                    waitg(*bufA, semGA)
                    compute(g, base, *bufA, msgA, semSA)

                    @pl.when(g + 2 < NGC)
                    def _():
                        fire(g + 2, e0k, *bufA, semGA)
                    waitg(*bufB, semGB)
                    compute(g + 1, base, *bufB, msgB, semSB)
                    return 0
                lax.fori_loop(0, NGC // 2, pair, 0)
                return 0
            lax.fori_loop(0, NCH, chunk, 0)

            pltpu.make_async_copy(msgA, acc_sh.at[gM], semSA).wait()
            pltpu.make_async_copy(msgB, acc_sh.at[gM], semSB).wait()
            plsc.subcore_barrier()

            def rok(k, _):
                row0 = sid * RT + k * 16
                pltpu.sync_copy(acc_sh.at[pl.ds(row0, 16)], msgA)
                pltpu.sync_copy(msgA, out_hbm.at[pl.ds(base + row0, 16)])
                return 0
            lax.fori_loop(0, NZ, rok, 0)
            plsc.subcore_barrier()

    return pl.kernel(
        body,
        out_type=jax.ShapeDtypeStruct((_NP, W), jnp.float32),
        mesh=mesh,
        compiler_params=pltpu.CompilerParams(use_tc_tiling_on_sc=False),
        scratch_types=(
            pltpu.VMEM_SHARED((M + 16, W), jnp.float32),
            pltpu.VMEM((CH,), jnp.int32),
            pltpu.VMEM((CH,), jnp.int32),
            pltpu.VMEM((16, W), jnp.float32),
            pltpu.VMEM((16, W), jnp.float32),
            pltpu.VMEM((16, 16), jnp.float32),
            pltpu.VMEM((16, 16), jnp.float32),
            pltpu.VMEM((16, 16), jnp.float32),
            pltpu.VMEM((16, 16), jnp.float32),
            pltpu.VMEM((16, 16), jnp.float32),
            pltpu.VMEM((16, 16), jnp.float32),
            pltpu.VMEM((16, W), jnp.float32),
            pltpu.VMEM((16, W), jnp.float32),
            pltpu.SemaphoreType.DMA,
            pltpu.SemaphoreType.DMA,
            pltpu.SemaphoreType.DMA,
            pltpu.SemaphoreType.DMA,
        ),
    )(feat, src, dst, ex, s0, s1)


def kernel(h, edge_index, W1, attn_l1, attn_r1, b1, W2, attn_l2, attn_r2, b2):
    src = edge_index[0]
    dst = edge_index[1]
    h_pad = jnp.zeros((_NP, _IN), jnp.float32).at[:_N].set(h)

    eye = jnp.eye(_HEADS, dtype=jnp.float32)
    Al = jnp.einsum("hd,hk->hdk", attn_l1[0], eye).reshape(_HEADS * _HID, _HEADS)
    Ar = jnp.einsum("hd,hk->hdk", attn_r1[0], eye).reshape(_HEADS * _HID, _HEADS)
    AL1 = jnp.concatenate([Al, Al], axis=1)
    AR1 = jnp.concatenate([Ar, Ar], axis=1)
    AL2 = jnp.broadcast_to(attn_l2[0, 0][:, None], (_OUT, 16))
    AR2 = jnp.broadcast_to(attn_r2[0, 0][:, None], (_OUT, 16))

    feat1, eL1, eR1 = _tc_feat(h_pad, W1, AL1, AR1)
    ex1, sp1 = _sc_stats(eL1, eR1, src, dst)
    out1 = _sc_agg(_HEADS * _HID, 2560, 2, _HEADS,
                   feat1, src, dst, ex1, sp1[:_NP], sp1[_NP:])

    feat2, eL2, eR2 = _tc_feat(out1, W2, AL2, AR2, elu_bias=b1)
    ex2, sp2 = _sc_stats(eL2, eR2, src, dst)
    out2 = _sc_agg(_OUT, 5120, 1, 1,
                   feat2, src, dst, ex2, sp2[:_NP], sp2[_NP:])

    final = _tc_bias(out2, b2)
    return final[:_N]


# speedup vs baseline: 13.2336x; 2.1960x over previous
import functools

import jax
import jax.numpy as jnp
from jax import lax
from jax.experimental import pallas as pl
from jax.experimental.pallas import tpu as pltpu
from jax.experimental.pallas import tpu_sc as plsc

_N = 10000
_NP = 10240
_E = 320000
_IN = 128
_HID = 64
_OUT = 128
_HEADS = 8

_NC = 2
_NS = 16
_G = 80


def _tc_feat(x, W, AL, AR, elu_bias=None):
    NPAD, K = x.shape
    M = W.shape[1]
    BN = 1280
    grid = (NPAD // BN,)
    with_elu = elu_bias is not None

    def body(*refs):
        if with_elu:
            x_ref, b_ref, w_ref, al_ref, ar_ref, f_ref, l_ref, r_ref = refs
            xv = x_ref[...] + b_ref[...]
            xv = jnp.where(xv > 0, xv, jnp.exp(xv) - 1.0)
        else:
            x_ref, w_ref, al_ref, ar_ref, f_ref, l_ref, r_ref = refs
            xv = x_ref[...]
        f = jnp.dot(xv, w_ref[...], preferred_element_type=jnp.float32)
        f_ref[...] = f
        l_ref[...] = jnp.dot(f, al_ref[...], preferred_element_type=jnp.float32)
        r_ref[...] = jnp.dot(f, ar_ref[...], preferred_element_type=jnp.float32)

    in_specs = [pl.BlockSpec((BN, K), lambda i: (i, 0))]
    args = [x]
    if with_elu:
        in_specs.append(pl.BlockSpec((1, K), lambda i: (0, 0)))
        args.append(elu_bias.reshape(1, K))
    in_specs += [
        pl.BlockSpec((K, M), lambda i: (0, 0)),
        pl.BlockSpec((M, 16), lambda i: (0, 0)),
        pl.BlockSpec((M, 16), lambda i: (0, 0)),
    ]
    args += [W, AL, AR]
    return pl.pallas_call(
        body,
        grid=grid,
        in_specs=in_specs,
        out_specs=[
            pl.BlockSpec((BN, M), lambda i: (i, 0)),
            pl.BlockSpec((BN, 16), lambda i: (i, 0)),
            pl.BlockSpec((BN, 16), lambda i: (i, 0)),
        ],
        out_shape=[
            jax.ShapeDtypeStruct((NPAD, M), jnp.float32),
            jax.ShapeDtypeStruct((NPAD, 16), jnp.float32),
            jax.ShapeDtypeStruct((NPAD, 16), jnp.float32),
        ],
    )(*args)


def _tc_bias(x, b):
    NPAD, M = x.shape
    BN = 1280

    def body(x_ref, b_ref, o_ref):
        o_ref[...] = x_ref[...] + b_ref[...]

    return pl.pallas_call(
        body,
        grid=(NPAD // BN,),
        in_specs=[pl.BlockSpec((BN, M), lambda i: (i, 0)),
                  pl.BlockSpec((1, M), lambda i: (0, 0))],
        out_specs=pl.BlockSpec((BN, M), lambda i: (i, 0)),
        out_shape=jax.ShapeDtypeStruct((NPAD, M), jnp.float32),
    )(x, b.reshape(1, M))


def _sc_stats(eL, eR, src, dst):
    EPT = _E // (_NC * _NS)
    NG = EPT // _G
    RT = _NP // _NS
    mesh = plsc.VectorSubcoreMesh(core_axis_name="c", subcore_axis_name="s")

    def body(eL_hbm, eR_hbm, src_hbm, dst_hbm, ex_hbm, sp_hbm,
             s_sh, src_i, dst_i, lrows, rrows, exs, zbuf):
        cid = lax.axis_index("c")
        sid = lax.axis_index("s")
        wid = cid * _NS + sid
        z16 = jnp.zeros((16,), jnp.float32)

        def zrow(i, _):
            zbuf[i, :] = z16
            return 0
        lax.fori_loop(0, RT, zrow, 0)
        pltpu.sync_copy(zbuf, s_sh.at[pl.ds(sid * RT, RT)])
        plsc.subcore_barrier()

        def group(g, _):
            e0 = wid * EPT + g * _G
            pltpu.sync_copy(src_hbm.at[pl.ds(e0, _G)], src_i)
            pltpu.sync_copy(dst_hbm.at[pl.ds(e0, _G)], dst_i)
            pltpu.sync_copy(eL_hbm.at[src_i], lrows)
            pltpu.sync_copy(eR_hbm.at[dst_i], rrows)

            def erow(e, _):
                v = lrows[e, :] + rrows[e, :]
                v = jnp.where(v >= 0, v, 0.2 * v)
                exs[e, :] = jnp.exp(v)
                return 0
            lax.fori_loop(0, _G, erow, 0)
            pltpu.sync_copy(exs, ex_hbm.at[pl.ds(e0, _G)])
            pltpu.sync_copy(exs, s_sh.at[dst_i], add=True)
            return 0
        lax.fori_loop(0, NG, group, 0)
        plsc.subcore_barrier()

        pltpu.sync_copy(s_sh.at[pl.ds(sid * RT, RT)], zbuf)
        pltpu.sync_copy(zbuf, sp_hbm.at[pl.ds(cid * _NP + sid * RT, RT)])

    return pl.kernel(
        body,
        out_type=(jax.ShapeDtypeStruct((_E, 16), jnp.float32),
                  jax.ShapeDtypeStruct((2 * _NP, 16), jnp.float32)),
        mesh=mesh,
        compiler_params=pltpu.CompilerParams(use_tc_tiling_on_sc=False,
                                             needs_layout_passes=False),
        scratch_types=(
            pltpu.VMEM_SHARED((_NP, 16), jnp.float32),
            pltpu.VMEM((_G,), jnp.int32),
            pltpu.VMEM((_G,), jnp.int32),
            pltpu.VMEM((_G, 16), jnp.float32),
            pltpu.VMEM((_G, 16), jnp.float32),
            pltpu.VMEM((_G, 16), jnp.float32),
            pltpu.VMEM((RT, 16), jnp.float32),
        ),
    )(eL, eR, src, dst)


_NRANGE = 4
_RSZ = _NP // _NRANGE
_REC = 64
_NRECCAP = 157
_BSTRIDE = _NRECCAP * 3 * _REC

def _sc_bucket(src, dst):
    EPT = _E // (_NC * _NS)
    CH = 2000
    NCH = EPT // CH
    NST = CH // 16
    mesh = plsc.VectorSubcoreMesh(core_axis_name="c", subcore_axis_name="s")

    def body(src_hbm, dst_hbm, lists_hbm, cnt_hbm,
             src_big, dst_big, cb, cntb):
        cid = lax.axis_index("c")
        sid = lax.axis_index("s")
        wid = cid * _NS + sid
        lane = jax.lax.iota(jnp.int32, 16)
        zsrc = jnp.zeros((16,), jnp.int32)
        zdst = jnp.full((16,), _NP - 1, jnp.int32)

        def flush(r, o, wr):
            b0 = (wid * _NRANGE + r) * _BSTRIDE + wr * 3 * _REC
            pltpu.sync_copy(cb.at[3 * r + 0, pl.ds(0, _REC)],
                            lists_hbm.at[pl.ds(b0, _REC)])
            pltpu.sync_copy(cb.at[3 * r + 1, pl.ds(0, _REC)],
                            lists_hbm.at[pl.ds(b0 + _REC, _REC)])
            pltpu.sync_copy(cb.at[3 * r + 2, pl.ds(0, _REC)],
                            lists_hbm.at[pl.ds(b0 + 2 * _REC, _REC)])
            for l in range(3):
                v = cb[3 * r + l, pl.ds(_REC, 16)]
                cb[3 * r + l, pl.ds(0, 16)] = v

        def chunk(k, carry):
            e0k = wid * EPT + k * CH
            pltpu.sync_copy(src_hbm.at[pl.ds(e0k, CH)], src_big)
            pltpu.sync_copy(dst_hbm.at[pl.ds(e0k, CH)], dst_big)

            def step(st, carry):
                svec = src_big[pl.ds(st * 16, 16)]
                dvec = dst_big[pl.ds(st * 16, 16)]
                evec = jnp.full((16,), e0k + st * 16, jnp.int32) + lane
                new = []
                for r in range(_NRANGE):
                    o, wr = carry[2 * r], carry[2 * r + 1]
                    msk = (dvec >= r * _RSZ) & (dvec < (r + 1) * _RSZ)
                    pc = plsc.all_reduce_population_count(msk)[0]
                    plsc.store_compressed(cb.at[3 * r + 0].at[pl.ds(o, 16)],
                                          svec, mask=msk)
                    plsc.store_compressed(cb.at[3 * r + 1].at[pl.ds(o, 16)],
                                          dvec, mask=msk)
                    plsc.store_compressed(cb.at[3 * r + 2].at[pl.ds(o, 16)],
                                          evec, mask=msk)
                    o = o + pc

                    @pl.when(o >= _REC)
                    def _():
                        flush(r, o, wr)
                    wr = jnp.where(o >= _REC, wr + 1, wr)
                    o = jnp.where(o >= _REC, o - _REC, o)
                    new += [o, wr]
                return tuple(new)
            return lax.fori_loop(0, NST, step, carry)
        carry = lax.fori_loop(0, NCH, chunk, (0, 0) * _NRANGE)

        cv = jnp.zeros((16,), jnp.int32)
        for r in range(_NRANGE):
            o, wr = carry[2 * r], carry[2 * r + 1]
            for j in range(4):
                cb[3 * r + 0, pl.ds(o + 16 * j, 16)] = zsrc
                cb[3 * r + 1, pl.ds(o + 16 * j, 16)] = zdst
                cb[3 * r + 2, pl.ds(o + 16 * j, 16)] = zsrc

            @pl.when(o > 0)
            def _():
                flush(r, o, wr)
            wr = jnp.where(o > 0, wr + 1, wr)
            cv = jnp.where(lane == r, wr * _REC, cv)
        cntb[...] = cv
        pltpu.sync_copy(cntb, cnt_hbm.at[pl.ds(wid * 16, 16)])

    return pl.kernel(
        body,
        out_type=(jax.ShapeDtypeStruct((_NC * _NS * _NRANGE * _BSTRIDE,),
                                       jnp.int32),
                  jax.ShapeDtypeStruct((_NC * _NS * 16,), jnp.int32)),
        mesh=mesh,
        compiler_params=pltpu.CompilerParams(use_tc_tiling_on_sc=False,
                                             needs_layout_passes=False),
        scratch_types=(
            pltpu.VMEM((CH,), jnp.int32),
            pltpu.VMEM((CH,), jnp.int32),
            pltpu.VMEM((3 * _NRANGE, 128), jnp.int32),
            pltpu.VMEM((16,), jnp.int32),
        ),
    )(src, dst)


def _sc_agg(W, M, R, NH, feat, ex, s0, s1, lists, cnt):
    HIDW = W // NH
    NV = HIDW // 16
    NBR = M // _RSZ
    RT = M // _NS
    NZ = RT // 16
    mesh = plsc.VectorSubcoreMesh(core_axis_name="c", subcore_axis_name="s")

    def body(feat_hbm, ex_hbm, s0_hbm, s1_hbm, lists_hbm, cnt_hbm, out_hbm,
             acc_sh, recb, cntb, featA, featB, exA, exB, s0A, s0B,
             s1A, s1B, msgA, msgB, semGA, semGB, semSA, semSB):
        cid = lax.axis_index("c")
        sid = lax.axis_index("s")
        lane = jax.lax.iota(jnp.int32, 16)
        z16 = jnp.zeros((16,), jnp.float32)
        gM = jnp.full((16,), M, jnp.int32)

        def fire(g, featX, exX, s0X, s1X, semX):
            svec = recb[pl.ds(g * 16, 16)]
            dvec = recb[pl.ds(_REC + g * 16, 16)]
            evec = recb[pl.ds(2 * _REC + g * 16, 16)]
            pltpu.async_copy(feat_hbm.at[svec], featX, semX)
            pltpu.async_copy(ex_hbm.at[evec], exX, semX)
            pltpu.async_copy(s0_hbm.at[dvec], s0X, semX)
            pltpu.async_copy(s1_hbm.at[dvec], s1X, semX)

        def waitg(featX, exX, s0X, s1X, semX):
            pltpu.make_async_copy(feat_hbm.at[gM], featX, semX).wait()
            pltpu.make_async_copy(ex_hbm.at[gM], exX, semX).wait()
            pltpu.make_async_copy(s0_hbm.at[gM], s0X, semX).wait()
            pltpu.make_async_copy(s1_hbm.at[gM], s1X, semX).wait()

        def compute(g, base, featX, exX, s0X, s1X, msgX, semSX):
            dvec = recb[pl.ds(_REC + g * 16, 16)]
            inr = (dvec >= base) & (dvec < base + M)
            doff = jnp.where(inr, dvec - base, M)
            pltpu.make_async_copy(msgX, acc_sh.at[gM], semSX).wait()

            def ebody(t, _):
                ar = exX[t, :] / (s0X[t, :] + s1X[t, :])
                for h in range(NH):
                    a = ar[h]
                    for k in range(NV):
                        off = h * HIDW + k * 16
                        msgX[t, pl.ds(off, 16)] = (
                            a * featX[t, pl.ds(off, 16)])
                return 0
            lax.fori_loop(0, 16, ebody, 0)
            pltpu.async_copy(msgX, acc_sh.at[doff], semSX, add=True)

        bufA = (featA, exA, s0A, s1A)
        bufB = (featB, exB, s0B, s1B)

        for r in range(R):
            rid = cid * R + r
            base = rid * M

            def zrow(i, _):
                for k in range(W // 16):
                    msgA[i, pl.ds(k * 16, 16)] = z16
                    msgB[i, pl.ds(k * 16, 16)] = z16
                return 0
            lax.fori_loop(0, 16, zrow, 0)

            def initk(k, _):
                pltpu.sync_copy(msgA, acc_sh.at[pl.ds(sid * RT + k * 16, 16)])
                return 0
            lax.fori_loop(0, NZ, initk, 0)
            plsc.subcore_barrier()

            pltpu.async_copy(msgA, acc_sh.at[gM], semSA, add=True)
            pltpu.async_copy(msgB, acc_sh.at[gM], semSB, add=True)

            for jj in range(2):
                j = 2 * sid + jj
                for br in range(NBR):
                    brid = rid * NBR + br
                    pltpu.sync_copy(cnt_hbm.at[pl.ds(j * 16, 16)], cntb)
                    cv = cntb[...]
                    cntv = jnp.sum(jnp.where(lane == brid, cv, 0))
                    nrec = cntv // _REC
                    bbase = (j * _NRANGE + brid) * _BSTRIDE

                    def rec_body(rec, _):
                        pltpu.sync_copy(
                            lists_hbm.at[pl.ds(bbase + rec * 3 * _REC,
                                               3 * _REC)], recb)
                        fire(0, *bufA, semGA)
                        fire(1, *bufB, semGB)
                        waitg(*bufA, semGA)
                        compute(0, base, *bufA, msgA, semSA)
                        fire(2, *bufA, semGA)
                        waitg(*bufB, semGB)
                        compute(1, base, *bufB, msgB, semSB)
                        fire(3, *bufB, semGB)
                        waitg(*bufA, semGA)
                        compute(2, base, *bufA, msgA, semSA)
                        waitg(*bufB, semGB)
                        compute(3, base, *bufB, msgB, semSB)
                        return 0
                    lax.fori_loop(0, nrec, rec_body, 0)

            pltpu.make_async_copy(msgA, acc_sh.at[gM], semSA).wait()
            pltpu.make_async_copy(msgB, acc_sh.at[gM], semSB).wait()
            plsc.subcore_barrier()

            def rok(k, _):
                row0 = sid * RT + k * 16
                pltpu.sync_copy(acc_sh.at[pl.ds(row0, 16)], msgA)
                pltpu.sync_copy(msgA, out_hbm.at[pl.ds(base + row0, 16)])
                return 0
            lax.fori_loop(0, NZ, rok, 0)
            plsc.subcore_barrier()

    return pl.kernel(
        body,
        out_type=jax.ShapeDtypeStruct((_NP, W), jnp.float32),
        mesh=mesh,
        compiler_params=pltpu.CompilerParams(use_tc_tiling_on_sc=False,
                                             needs_layout_passes=False),
        scratch_types=(
            pltpu.VMEM_SHARED((M + 16, W), jnp.float32),
            pltpu.VMEM((3 * _REC,), jnp.int32),
            pltpu.VMEM((16,), jnp.int32),
            pltpu.VMEM((16, W), jnp.float32),
            pltpu.VMEM((16, W), jnp.float32),
            pltpu.VMEM((16, 16), jnp.float32),
            pltpu.VMEM((16, 16), jnp.float32),
            pltpu.VMEM((16, 16), jnp.float32),
            pltpu.VMEM((16, 16), jnp.float32),
            pltpu.VMEM((16, 16), jnp.float32),
            pltpu.VMEM((16, 16), jnp.float32),
            pltpu.VMEM((16, W), jnp.float32),
            pltpu.VMEM((16, W), jnp.float32),
            pltpu.SemaphoreType.DMA,
            pltpu.SemaphoreType.DMA,
            pltpu.SemaphoreType.DMA,
            pltpu.SemaphoreType.DMA,
        ),
    )(feat, ex, s0, s1, lists, cnt)


def kernel(h, edge_index, W1, attn_l1, attn_r1, b1, W2, attn_l2, attn_r2, b2):
    src = edge_index[0]
    dst = edge_index[1]
    h_pad = jnp.zeros((_NP, _IN), jnp.float32).at[:_N].set(h)

    eye = jnp.eye(_HEADS, dtype=jnp.float32)
    Al = jnp.einsum("hd,hk->hdk", attn_l1[0], eye).reshape(_HEADS * _HID, _HEADS)
    Ar = jnp.einsum("hd,hk->hdk", attn_r1[0], eye).reshape(_HEADS * _HID, _HEADS)
    AL1 = jnp.concatenate([Al, Al], axis=1)
    AR1 = jnp.concatenate([Ar, Ar], axis=1)
    AL2 = jnp.broadcast_to(attn_l2[0, 0][:, None], (_OUT, 16))
    AR2 = jnp.broadcast_to(attn_r2[0, 0][:, None], (_OUT, 16))

    lists, cnt = _sc_bucket(src, dst)

    feat1, eL1, eR1 = _tc_feat(h_pad, W1, AL1, AR1)
    ex1, sp1 = _sc_stats(eL1, eR1, src, dst)
    out1 = _sc_agg(_HEADS * _HID, 2560, 2, _HEADS,
                   feat1, ex1, sp1[:_NP], sp1[_NP:], lists, cnt)

    feat2, eL2, eR2 = _tc_feat(out1, W2, AL2, AR2, elu_bias=b1)
    ex2, sp2 = _sc_stats(eL2, eR2, src, dst)
    out2 = _sc_agg(_OUT, 5120, 1, 1,
                   feat2, ex2, sp2[:_NP], sp2[_NP:], lists, cnt)

    final = _tc_bias(out2, b2)
    return final[:_N]
